# Initial kernel scaffold; baseline (speedup 1.0000x reference)
import jax
import jax.numpy as jnp
from jax.experimental import pallas as pl


def kernel(x, edge_index, node_graph_ids, W1, b1, W2, b2, Wc1, bc1, Wc2, bc2):
    raise NotImplementedError("write your pallas kernel here")


# speedup vs baseline: 8.3374x; 8.3374x over previous
import functools

import jax
import jax.numpy as jnp
from jax import lax
from jax.experimental import pallas as pl
from jax.experimental.pallas import tpu as pltpu
from jax.experimental.pallas import tpu_sc as plsc

N_NODES = 10000
D_HID = 32
N_GRAPHS = 64
N_CLS = 10

NC, NS, L = 2, 16, 16
NW = NC * NS
CHUNK = 128
ROWS_PER_W = 80
E_PAD = NW * ROWS_PER_W * CHUNK
N_ACC = 10112
ROWS_PER_TILE = N_ACC // NS

BLK = 2000
N_BLKS = N_NODES // BLK


def _proj_body(x_ref, w_ref, o_ref):
    o_ref[...] = jnp.dot(x_ref[...], w_ref[...],
                         preferred_element_type=jnp.float32)


def _project(x, W):
    n, k = x.shape
    m = W.shape[1]
    return pl.pallas_call(
        _proj_body,
        grid=(n // BLK,),
        in_specs=[pl.BlockSpec((BLK, k), lambda i: (i, 0)),
                  pl.BlockSpec((k, m), lambda i: (0, 0))],
        out_specs=pl.BlockSpec((BLK, m), lambda i: (i, 0)),
        out_shape=jax.ShapeDtypeStruct((n, m), jnp.float32),
    )(x, W)


def _segsum_partials(p, src2d, dst2d):
    mesh = plsc.VectorSubcoreMesh(core_axis_name="c", subcore_axis_name="s",
                                  num_cores=NC, num_subcores=NS)

    @functools.partial(
        pl.kernel,
        out_type=jax.ShapeDtypeStruct((NC, N_ACC, D_HID), jnp.float32),
        mesh=mesh,
        scratch_types=[
            pltpu.VMEM((ROWS_PER_W, CHUNK), jnp.int32),
            pltpu.VMEM((ROWS_PER_W, CHUNK), jnp.int32),
            pltpu.VMEM((CHUNK, D_HID), jnp.float32),
            pltpu.VMEM((ROWS_PER_TILE, D_HID), jnp.float32),
            pltpu.VMEM_SHARED((N_ACC, D_HID), jnp.float32),
            pltpu.SemaphoreType.DMA,
        ],
        compiler_params=pltpu.CompilerParams(use_tc_tiling_on_sc=False),
    )
    def seg_kernel(p_hbm, src_hbm, dst_hbm, out_hbm,
                   src_v, dst_v, rows_v, tbuf, acc, sem):
        c = lax.axis_index("c")
        s = lax.axis_index("s")
        wid = s * NC + c
        row0 = s * ROWS_PER_TILE

        zero = jnp.zeros((L,), jnp.float32)

        def zloop(r, carry):
            tbuf[r, pl.ds(0, L)] = zero
            tbuf[r, pl.ds(L, L)] = zero
            return carry

        lax.fori_loop(0, ROWS_PER_TILE, zloop, 0, unroll=4)
        pltpu.sync_copy(tbuf, acc.at[pl.ds(row0, ROWS_PER_TILE)])
        plsc.subcore_barrier()

        pltpu.sync_copy(src_hbm.at[pl.ds(wid * ROWS_PER_W, ROWS_PER_W)], src_v)
        pltpu.sync_copy(dst_hbm.at[pl.ds(wid * ROWS_PER_W, ROWS_PER_W)], dst_v)

        def eloop(j, carry):
            pltpu.async_copy(p_hbm.at[src_v.at[j]], rows_v, sem).wait()
            pltpu.sync_copy(rows_v, acc.at[dst_v.at[j]], add=True)
            return carry

        lax.fori_loop(0, ROWS_PER_W, eloop, 0)
        plsc.subcore_barrier()

        pltpu.sync_copy(acc.at[pl.ds(row0, ROWS_PER_TILE)], tbuf)
        pltpu.sync_copy(tbuf, out_hbm.at[c, pl.ds(row0, ROWS_PER_TILE)])

    return seg_kernel(p, src2d, dst2d)


def _combine_body(p_ref, parts_ref, b_ref, w_ref, o_ref):
    h = p_ref[...] + parts_ref[0] + parts_ref[1] + b_ref[...]
    h = jnp.maximum(h, 0.0)
    o_ref[...] = jnp.dot(h, w_ref[...], preferred_element_type=jnp.float32)


def _combine_project(p, parts, b, W):
    return pl.pallas_call(
        _combine_body,
        grid=(N_BLKS,),
        in_specs=[pl.BlockSpec((BLK, D_HID), lambda i: (i, 0)),
                  pl.BlockSpec((NC, BLK, D_HID), lambda i: (0, i, 0)),
                  pl.BlockSpec((1, D_HID), lambda i: (0, 0)),
                  pl.BlockSpec((D_HID, D_HID), lambda i: (0, 0))],
        out_specs=pl.BlockSpec((BLK, D_HID), lambda i: (i, 0)),
        out_shape=jax.ShapeDtypeStruct((N_NODES, D_HID), jnp.float32),
    )(p, parts, b, W)


def _pool_mlp_body(p_ref, parts_ref, b_ref, gid_ref,
                   wc1_ref, bc1_ref, wc2_ref, bc2_ref,
                   o_ref, sums, counts):
    i = pl.program_id(0)
    h = p_ref[...] + parts_ref[0] + parts_ref[1] + b_ref[...]
    h = jnp.maximum(h, 0.0)
    gid = gid_ref[...]
    onehot = (gid == lax.broadcasted_iota(jnp.int32, (BLK, N_GRAPHS), 1))
    onehot = onehot.astype(jnp.float32)
    dn = (((0,), (0,)), ((), ()))
    blk_sums = lax.dot_general(onehot, h, dn,
                               preferred_element_type=jnp.float32)
    blk_cnts = lax.dot_general(onehot, jnp.ones_like(h), dn,
                               preferred_element_type=jnp.float32)

    @pl.when(i == 0)
    def _():
        sums[...] = jnp.zeros_like(sums)
        counts[...] = jnp.zeros_like(counts)

    sums[...] += blk_sums
    counts[...] += blk_cnts

    @pl.when(i == N_BLKS - 1)
    def _():
        h_g = sums[...] / jnp.maximum(counts[...], 1.0)
        hid = jnp.dot(h_g, wc1_ref[...],
                      preferred_element_type=jnp.float32) + bc1_ref[...]
        hid = jnp.maximum(hid, 0.0)
        o_ref[...] = jnp.dot(hid, wc2_ref[...],
                             preferred_element_type=jnp.float32) + bc2_ref[...]


def _pool_mlp(p, parts, b, gids, Wc1, bc1, Wc2, bc2):
    return pl.pallas_call(
        _pool_mlp_body,
        grid=(N_BLKS,),
        in_specs=[pl.BlockSpec((BLK, D_HID), lambda i: (i, 0)),
                  pl.BlockSpec((NC, BLK, D_HID), lambda i: (0, i, 0)),
                  pl.BlockSpec((1, D_HID), lambda i: (0, 0)),
                  pl.BlockSpec((BLK, 1), lambda i: (i, 0)),
                  pl.BlockSpec((D_HID, D_HID), lambda i: (0, 0)),
                  pl.BlockSpec((1, D_HID), lambda i: (0, 0)),
                  pl.BlockSpec((D_HID, N_CLS), lambda i: (0, 0)),
                  pl.BlockSpec((1, N_CLS), lambda i: (0, 0))],
        out_specs=pl.BlockSpec((N_GRAPHS, N_CLS), lambda i: (0, 0)),
        out_shape=jax.ShapeDtypeStruct((N_GRAPHS, N_CLS), jnp.float32),
        scratch_shapes=[pltpu.VMEM((N_GRAPHS, D_HID), jnp.float32),
                        pltpu.VMEM((N_GRAPHS, D_HID), jnp.float32)],
    )(p, parts, b, gids, Wc1, bc1, Wc2, bc2)


def kernel(x, edge_index, node_graph_ids, W1, b1, W2, b2, Wc1, bc1, Wc2, bc2):
    src = edge_index[0].astype(jnp.int32)
    dst = edge_index[1].astype(jnp.int32)
    pad = E_PAD - src.shape[0]
    src2d = jnp.concatenate(
        [src, jnp.zeros((pad,), jnp.int32)]).reshape(E_PAD // CHUNK, CHUNK)
    dst2d = jnp.concatenate(
        [dst, jnp.full((pad,), N_NODES, jnp.int32)]).reshape(
            E_PAD // CHUNK, CHUNK)
    gids = node_graph_ids.astype(jnp.int32).reshape(N_NODES, 1)

    p1 = _project(x, W1)
    parts1 = _segsum_partials(p1, src2d, dst2d)
    p2 = _combine_project(p1, parts1, b1.reshape(1, -1), W2)
    parts2 = _segsum_partials(p2, src2d, dst2d)
    return _pool_mlp(p2, parts2, b2.reshape(1, -1), gids,
                     Wc1, bc1.reshape(1, -1), Wc2, bc2.reshape(1, -1))


# speedup vs baseline: 9.0276x; 1.0828x over previous
import functools

import jax
import jax.numpy as jnp
from jax import lax
from jax.experimental import pallas as pl
from jax.experimental.pallas import tpu as pltpu
from jax.experimental.pallas import tpu_sc as plsc

N_NODES = 10000
D_HID = 32
N_GRAPHS = 64
N_CLS = 10

NC, NS, L = 2, 16, 16
NW = NC * NS
CHUNK = 128
ROWS_PER_W = 80
E_PAD = NW * ROWS_PER_W * CHUNK
N_ACC = 10112
ROWS_PER_TILE = N_ACC // NS

BLK = 2000
N_BLKS = N_NODES // BLK


def _proj_body(x_ref, w_ref, o_ref):
    o_ref[...] = jnp.dot(x_ref[...], w_ref[...],
                         preferred_element_type=jnp.float32)


def _project(x, W):
    n, k = x.shape
    m = W.shape[1]
    return pl.pallas_call(
        _proj_body,
        grid=(n // BLK,),
        in_specs=[pl.BlockSpec((BLK, k), lambda i: (i, 0)),
                  pl.BlockSpec((k, m), lambda i: (0, 0))],
        out_specs=pl.BlockSpec((BLK, m), lambda i: (i, 0)),
        out_shape=jax.ShapeDtypeStruct((n, m), jnp.float32),
    )(x, W)


def _segsum_partials(p, src2d, dst2d):
    mesh = plsc.VectorSubcoreMesh(core_axis_name="c", subcore_axis_name="s",
                                  num_cores=NC, num_subcores=NS)

    @functools.partial(
        pl.kernel,
        out_type=jax.ShapeDtypeStruct((NC, N_ACC, D_HID), jnp.float32),
        mesh=mesh,
        scratch_types=[
            pltpu.VMEM((ROWS_PER_W, CHUNK), jnp.int32),
            pltpu.VMEM((ROWS_PER_W, CHUNK), jnp.int32),
            pltpu.VMEM((CHUNK, D_HID), jnp.float32),
            pltpu.VMEM((CHUNK, D_HID), jnp.float32),
            pltpu.VMEM((ROWS_PER_TILE, D_HID), jnp.float32),
            pltpu.VMEM_SHARED((N_ACC, D_HID), jnp.float32),
            pltpu.SemaphoreType.DMA,
            pltpu.SemaphoreType.DMA,
        ],
        compiler_params=pltpu.CompilerParams(use_tc_tiling_on_sc=False),
    )
    def seg_kernel(p_hbm, src_hbm, dst_hbm, out_hbm,
                   src_v, dst_v, rows_a, rows_b, tbuf, acc, sem_a, sem_b):
        c = lax.axis_index("c")
        s = lax.axis_index("s")
        wid = s * NC + c
        row0 = s * ROWS_PER_TILE

        zero = jnp.zeros((L,), jnp.float32)

        def zloop(r, carry):
            tbuf[r, pl.ds(0, L)] = zero
            tbuf[r, pl.ds(L, L)] = zero
            return carry

        lax.fori_loop(0, ROWS_PER_TILE, zloop, 0, unroll=4)
        pltpu.sync_copy(tbuf, acc.at[pl.ds(row0, ROWS_PER_TILE)])
        plsc.subcore_barrier()

        pltpu.sync_copy(src_hbm.at[pl.ds(wid * ROWS_PER_W, ROWS_PER_W)], src_v)
        pltpu.sync_copy(dst_hbm.at[pl.ds(wid * ROWS_PER_W, ROWS_PER_W)], dst_v)

        bufs = (rows_a, rows_b)
        sems = (sem_a, sem_b)
        pltpu.async_copy(p_hbm.at[src_v.at[0]], rows_a, sem_a)

        def eloop(i, carry):
            for b in range(2):
                j = 2 * i + b
                nb = 1 - b
                pltpu.make_async_copy(p_hbm.at[src_v.at[j]],
                                      bufs[b], sems[b]).wait()

                @pl.when(j + 1 < ROWS_PER_W)
                def _():
                    pltpu.async_copy(p_hbm.at[src_v.at[j + 1]],
                                     bufs[nb], sems[nb])

                pltpu.sync_copy(bufs[b], acc.at[dst_v.at[j]], add=True)
            return carry

        lax.fori_loop(0, ROWS_PER_W // 2, eloop, 0)
        plsc.subcore_barrier()

        pltpu.sync_copy(acc.at[pl.ds(row0, ROWS_PER_TILE)], tbuf)
        pltpu.sync_copy(tbuf, out_hbm.at[c, pl.ds(row0, ROWS_PER_TILE)])

    return seg_kernel(p, src2d, dst2d)


def _combine_body(p_ref, parts_ref, b_ref, w_ref, o_ref):
    h = p_ref[...] + parts_ref[0] + parts_ref[1] + b_ref[...]
    h = jnp.maximum(h, 0.0)
    o_ref[...] = jnp.dot(h, w_ref[...], preferred_element_type=jnp.float32)


def _combine_project(p, parts, b, W):
    return pl.pallas_call(
        _combine_body,
        grid=(N_BLKS,),
        in_specs=[pl.BlockSpec((BLK, D_HID), lambda i: (i, 0)),
                  pl.BlockSpec((NC, BLK, D_HID), lambda i: (0, i, 0)),
                  pl.BlockSpec((1, D_HID), lambda i: (0, 0)),
                  pl.BlockSpec((D_HID, D_HID), lambda i: (0, 0))],
        out_specs=pl.BlockSpec((BLK, D_HID), lambda i: (i, 0)),
        out_shape=jax.ShapeDtypeStruct((N_NODES, D_HID), jnp.float32),
    )(p, parts, b, W)


def _pool_mlp_body(p_ref, parts_ref, b_ref, gid_ref,
                   wc1_ref, bc1_ref, wc2_ref, bc2_ref,
                   o_ref, sums, counts):
    i = pl.program_id(0)
    h = p_ref[...] + parts_ref[0] + parts_ref[1] + b_ref[...]
    h = jnp.maximum(h, 0.0)
    gid = gid_ref[...]
    onehot = (gid == lax.broadcasted_iota(jnp.int32, (BLK, N_GRAPHS), 1))
    onehot = onehot.astype(jnp.float32)
    dn = (((0,), (0,)), ((), ()))
    blk_sums = lax.dot_general(onehot, h, dn,
                               preferred_element_type=jnp.float32)
    blk_cnts = lax.dot_general(onehot, jnp.ones_like(h), dn,
                               preferred_element_type=jnp.float32)

    @pl.when(i == 0)
    def _():
        sums[...] = jnp.zeros_like(sums)
        counts[...] = jnp.zeros_like(counts)

    sums[...] += blk_sums
    counts[...] += blk_cnts

    @pl.when(i == N_BLKS - 1)
    def _():
        h_g = sums[...] / jnp.maximum(counts[...], 1.0)
        hid = jnp.dot(h_g, wc1_ref[...],
                      preferred_element_type=jnp.float32) + bc1_ref[...]
        hid = jnp.maximum(hid, 0.0)
        o_ref[...] = jnp.dot(hid, wc2_ref[...],
                             preferred_element_type=jnp.float32) + bc2_ref[...]


def _pool_mlp(p, parts, b, gids, Wc1, bc1, Wc2, bc2):
    return pl.pallas_call(
        _pool_mlp_body,
        grid=(N_BLKS,),
        in_specs=[pl.BlockSpec((BLK, D_HID), lambda i: (i, 0)),
                  pl.BlockSpec((NC, BLK, D_HID), lambda i: (0, i, 0)),
                  pl.BlockSpec((1, D_HID), lambda i: (0, 0)),
                  pl.BlockSpec((BLK, 1), lambda i: (i, 0)),
                  pl.BlockSpec((D_HID, D_HID), lambda i: (0, 0)),
                  pl.BlockSpec((1, D_HID), lambda i: (0, 0)),
                  pl.BlockSpec((D_HID, N_CLS), lambda i: (0, 0)),
                  pl.BlockSpec((1, N_CLS), lambda i: (0, 0))],
        out_specs=pl.BlockSpec((N_GRAPHS, N_CLS), lambda i: (0, 0)),
        out_shape=jax.ShapeDtypeStruct((N_GRAPHS, N_CLS), jnp.float32),
        scratch_shapes=[pltpu.VMEM((N_GRAPHS, D_HID), jnp.float32),
                        pltpu.VMEM((N_GRAPHS, D_HID), jnp.float32)],
    )(p, parts, b, gids, Wc1, bc1, Wc2, bc2)


def kernel(x, edge_index, node_graph_ids, W1, b1, W2, b2, Wc1, bc1, Wc2, bc2):
    src = edge_index[0].astype(jnp.int32)
    dst = edge_index[1].astype(jnp.int32)
    pad = E_PAD - src.shape[0]
    src2d = jnp.concatenate(
        [src, jnp.zeros((pad,), jnp.int32)]).reshape(E_PAD // CHUNK, CHUNK)
    dst2d = jnp.concatenate(
        [dst, jnp.full((pad,), N_NODES, jnp.int32)]).reshape(
            E_PAD // CHUNK, CHUNK)
    gids = node_graph_ids.astype(jnp.int32).reshape(N_NODES, 1)

    p1 = _project(x, W1)
    parts1 = _segsum_partials(p1, src2d, dst2d)
    p2 = _combine_project(p1, parts1, b1.reshape(1, -1), W2)
    parts2 = _segsum_partials(p2, src2d, dst2d)
    return _pool_mlp(p2, parts2, b2.reshape(1, -1), gids,
                     Wc1, bc1.reshape(1, -1), Wc2, bc2.reshape(1, -1))


# speedup vs baseline: 19.7386x; 2.1865x over previous
import functools

import jax
import jax.numpy as jnp
from jax import lax
from jax.experimental import pallas as pl
from jax.experimental.pallas import tpu as pltpu
from jax.experimental.pallas import tpu_sc as plsc

N_NODES = 10000
D_HID = 32
N_GRAPHS = 64
N_CLS = 10

NC, NS, L = 2, 16, 16
NW = NC * NS
CHUNK = 128
ROWS_PER_W = 80
E_PAD = NW * ROWS_PER_W * CHUNK
N_ACC = 10112
ROWS_PER_TILE = N_ACC // NS

BLK = 2000
N_BLKS = N_NODES // BLK


def _proj_body(x_ref, w_ref, o_ref):
    o_ref[...] = jnp.dot(x_ref[...], w_ref[...],
                         preferred_element_type=jnp.float32)


def _project(x, W):
    n, k = x.shape
    m = W.shape[1]
    return pl.pallas_call(
        _proj_body,
        grid=(n // BLK,),
        in_specs=[pl.BlockSpec((BLK, k), lambda i: (i, 0)),
                  pl.BlockSpec((k, m), lambda i: (0, 0))],
        out_specs=pl.BlockSpec((BLK, m), lambda i: (i, 0)),
        out_shape=jax.ShapeDtypeStruct((N_ACC, m), jnp.float32),
    )(x, W)


def _segsum_partials(p, src2d, dst2d):
    mesh = plsc.VectorSubcoreMesh(core_axis_name="c", subcore_axis_name="s",
                                  num_cores=NC, num_subcores=NS)

    @functools.partial(
        pl.kernel,
        out_type=jax.ShapeDtypeStruct((NC, N_ACC, D_HID), jnp.float32),
        mesh=mesh,
        scratch_types=[
            pltpu.VMEM((ROWS_PER_W, CHUNK), jnp.int32),
            pltpu.VMEM((ROWS_PER_W, CHUNK), jnp.int32),
            pltpu.VMEM((CHUNK, D_HID), jnp.float32),
            pltpu.VMEM((CHUNK, D_HID), jnp.float32),
            pltpu.VMEM((ROWS_PER_TILE, D_HID), jnp.float32),
            pltpu.VMEM_SHARED((N_ACC, D_HID), jnp.float32),
            pltpu.VMEM_SHARED((N_ACC, D_HID), jnp.float32),
            pltpu.SemaphoreType.DMA,
            pltpu.SemaphoreType.DMA,
        ],
        compiler_params=pltpu.CompilerParams(use_tc_tiling_on_sc=False),
    )
    def seg_kernel(p_hbm, src_hbm, dst_hbm, out_hbm,
                   src_v, dst_v, rows_a, rows_b, tbuf, acc, p_sh,
                   sem_a, sem_b):
        c = lax.axis_index("c")
        s = lax.axis_index("s")
        wid = s * NC + c
        row0 = s * ROWS_PER_TILE

        pltpu.sync_copy(p_hbm.at[pl.ds(row0, ROWS_PER_TILE)], tbuf)
        pltpu.sync_copy(tbuf, p_sh.at[pl.ds(row0, ROWS_PER_TILE)])

        zero = jnp.zeros((L,), jnp.float32)

        def zloop(r, carry):
            tbuf[r, pl.ds(0, L)] = zero
            tbuf[r, pl.ds(L, L)] = zero
            return carry

        lax.fori_loop(0, ROWS_PER_TILE, zloop, 0, unroll=4)
        pltpu.sync_copy(tbuf, acc.at[pl.ds(row0, ROWS_PER_TILE)])
        plsc.subcore_barrier()

        pltpu.sync_copy(src_hbm.at[pl.ds(wid * ROWS_PER_W, ROWS_PER_W)], src_v)
        pltpu.sync_copy(dst_hbm.at[pl.ds(wid * ROWS_PER_W, ROWS_PER_W)], dst_v)

        bufs = (rows_a, rows_b)
        sems = (sem_a, sem_b)
        pltpu.async_copy(p_sh.at[src_v.at[0]], rows_a, sem_a)

        def eloop(i, carry):
            for b in range(2):
                j = 2 * i + b
                nb = 1 - b
                pltpu.make_async_copy(p_sh.at[src_v.at[j]],
                                      bufs[b], sems[b]).wait()

                @pl.when(j + 1 < ROWS_PER_W)
                def _():
                    pltpu.async_copy(p_sh.at[src_v.at[j + 1]],
                                     bufs[nb], sems[nb])

                pltpu.sync_copy(bufs[b], acc.at[dst_v.at[j]], add=True)
            return carry

        lax.fori_loop(0, ROWS_PER_W // 2, eloop, 0)
        plsc.subcore_barrier()

        pltpu.sync_copy(acc.at[pl.ds(row0, ROWS_PER_TILE)], tbuf)
        pltpu.sync_copy(tbuf, out_hbm.at[c, pl.ds(row0, ROWS_PER_TILE)])

    return seg_kernel(p, src2d, dst2d)


def _combine_body(p_ref, parts_ref, b_ref, w_ref, o_ref):
    h = p_ref[...] + parts_ref[0] + parts_ref[1] + b_ref[...]
    h = jnp.maximum(h, 0.0)
    o_ref[...] = jnp.dot(h, w_ref[...], preferred_element_type=jnp.float32)


def _combine_project(p, parts, b, W):
    return pl.pallas_call(
        _combine_body,
        grid=(N_BLKS,),
        in_specs=[pl.BlockSpec((BLK, D_HID), lambda i: (i, 0)),
                  pl.BlockSpec((NC, BLK, D_HID), lambda i: (0, i, 0)),
                  pl.BlockSpec((1, D_HID), lambda i: (0, 0)),
                  pl.BlockSpec((D_HID, D_HID), lambda i: (0, 0))],
        out_specs=pl.BlockSpec((BLK, D_HID), lambda i: (i, 0)),
        out_shape=jax.ShapeDtypeStruct((N_ACC, D_HID), jnp.float32),
    )(p, parts, b, W)


def _pool_mlp_body(p_ref, parts_ref, b_ref, gid_ref,
                   wc1_ref, bc1_ref, wc2_ref, bc2_ref,
                   o_ref, sums, counts):
    i = pl.program_id(0)
    h = p_ref[...] + parts_ref[0] + parts_ref[1] + b_ref[...]
    h = jnp.maximum(h, 0.0)
    gid = gid_ref[...]
    onehot = (gid == lax.broadcasted_iota(jnp.int32, (BLK, N_GRAPHS), 1))
    onehot = onehot.astype(jnp.float32)
    dn = (((0,), (0,)), ((), ()))
    blk_sums = lax.dot_general(onehot, h, dn,
                               preferred_element_type=jnp.float32)
    blk_cnts = lax.dot_general(onehot, jnp.ones_like(h), dn,
                               preferred_element_type=jnp.float32)

    @pl.when(i == 0)
    def _():
        sums[...] = jnp.zeros_like(sums)
        counts[...] = jnp.zeros_like(counts)

    sums[...] += blk_sums
    counts[...] += blk_cnts

    @pl.when(i == N_BLKS - 1)
    def _():
        h_g = sums[...] / jnp.maximum(counts[...], 1.0)
        hid = jnp.dot(h_g, wc1_ref[...],
                      preferred_element_type=jnp.float32) + bc1_ref[...]
        hid = jnp.maximum(hid, 0.0)
        o_ref[...] = jnp.dot(hid, wc2_ref[...],
                             preferred_element_type=jnp.float32) + bc2_ref[...]


def _pool_mlp(p, parts, b, gids, Wc1, bc1, Wc2, bc2):
    return pl.pallas_call(
        _pool_mlp_body,
        grid=(N_BLKS,),
        in_specs=[pl.BlockSpec((BLK, D_HID), lambda i: (i, 0)),
                  pl.BlockSpec((NC, BLK, D_HID), lambda i: (0, i, 0)),
                  pl.BlockSpec((1, D_HID), lambda i: (0, 0)),
                  pl.BlockSpec((BLK, 1), lambda i: (i, 0)),
                  pl.BlockSpec((D_HID, D_HID), lambda i: (0, 0)),
                  pl.BlockSpec((1, D_HID), lambda i: (0, 0)),
                  pl.BlockSpec((D_HID, N_CLS), lambda i: (0, 0)),
                  pl.BlockSpec((1, N_CLS), lambda i: (0, 0))],
        out_specs=pl.BlockSpec((N_GRAPHS, N_CLS), lambda i: (0, 0)),
        out_shape=jax.ShapeDtypeStruct((N_GRAPHS, N_CLS), jnp.float32),
        scratch_shapes=[pltpu.VMEM((N_GRAPHS, D_HID), jnp.float32),
                        pltpu.VMEM((N_GRAPHS, D_HID), jnp.float32)],
    )(p, parts, b, gids, Wc1, bc1, Wc2, bc2)


def kernel(x, edge_index, node_graph_ids, W1, b1, W2, b2, Wc1, bc1, Wc2, bc2):
    src = edge_index[0].astype(jnp.int32)
    dst = edge_index[1].astype(jnp.int32)
    pad = E_PAD - src.shape[0]
    src2d = jnp.concatenate(
        [src, jnp.zeros((pad,), jnp.int32)]).reshape(E_PAD // CHUNK, CHUNK)
    dst2d = jnp.concatenate(
        [dst, jnp.full((pad,), N_NODES, jnp.int32)]).reshape(
            E_PAD // CHUNK, CHUNK)
    gids = node_graph_ids.astype(jnp.int32).reshape(N_NODES, 1)

    p1 = _project(x, W1)
    parts1 = _segsum_partials(p1, src2d, dst2d)
    p2 = _combine_project(p1, parts1, b1.reshape(1, -1), W2)
    parts2 = _segsum_partials(p2, src2d, dst2d)
    return _pool_mlp(p2, parts2, b2.reshape(1, -1), gids,
                     Wc1, bc1.reshape(1, -1), Wc2, bc2.reshape(1, -1))


# speedup vs baseline: 20.7349x; 1.0505x over previous
import functools

import jax
import jax.numpy as jnp
from jax import lax
from jax.experimental import pallas as pl
from jax.experimental.pallas import tpu as pltpu
from jax.experimental.pallas import tpu_sc as plsc

N_NODES = 10000
D_HID = 32
N_GRAPHS = 64
N_CLS = 10

NC, NS, L = 2, 16, 16
NW = NC * NS
CHUNK = 128
E_ROWS = 2500
ROWS_LO = E_ROWS // NW
N_EXTRA = E_ROWS - ROWS_LO * NW
N_ACC = 10112
ROWS_PER_TILE = N_ACC // NS

BLK = 2000
N_BLKS = N_NODES // BLK


def _proj_body(x_ref, w_ref, o_ref):
    o_ref[...] = jnp.dot(x_ref[...], w_ref[...],
                         preferred_element_type=jnp.float32)


def _project(x, W):
    n, k = x.shape
    m = W.shape[1]
    return pl.pallas_call(
        _proj_body,
        grid=(n // BLK,),
        in_specs=[pl.BlockSpec((BLK, k), lambda i: (i, 0)),
                  pl.BlockSpec((k, m), lambda i: (0, 0))],
        out_specs=pl.BlockSpec((BLK, m), lambda i: (i, 0)),
        out_shape=jax.ShapeDtypeStruct((N_ACC, m), jnp.float32),
    )(x, W)


def _segsum_partials(p, edges3d):
    mesh = plsc.VectorSubcoreMesh(core_axis_name="c", subcore_axis_name="s",
                                  num_cores=NC, num_subcores=NS)

    @functools.partial(
        pl.kernel,
        out_type=jax.ShapeDtypeStruct((NC, N_ACC, D_HID), jnp.float32),
        mesh=mesh,
        scratch_types=[
            pltpu.VMEM((ROWS_LO + 1, CHUNK), jnp.int32),
            pltpu.VMEM((ROWS_LO + 1, CHUNK), jnp.int32),
            pltpu.VMEM((CHUNK, D_HID), jnp.float32),
            pltpu.VMEM((CHUNK, D_HID), jnp.float32),
            pltpu.VMEM((ROWS_PER_TILE, D_HID), jnp.float32),
            pltpu.VMEM_SHARED((N_ACC, D_HID), jnp.float32),
            pltpu.VMEM_SHARED((N_ACC, D_HID), jnp.float32),
            pltpu.SemaphoreType.DMA,
            pltpu.SemaphoreType.DMA,
        ],
        compiler_params=pltpu.CompilerParams(use_tc_tiling_on_sc=False),
    )
    def seg_kernel(p_hbm, e_hbm, out_hbm,
                   src_v, dst_v, rows_a, rows_b, tbuf, acc, p_sh,
                   sem_a, sem_b):
        c = lax.axis_index("c")
        s = lax.axis_index("s")
        wid = s * NC + c
        row0 = s * ROWS_PER_TILE

        pltpu.sync_copy(p_hbm.at[pl.ds(row0, ROWS_PER_TILE)], tbuf)
        pltpu.sync_copy(tbuf, p_sh.at[pl.ds(row0, ROWS_PER_TILE)])

        @pl.when(c == 0)
        def _():
            pltpu.sync_copy(tbuf, acc.at[pl.ds(row0, ROWS_PER_TILE)])

        zero = jnp.zeros((L,), jnp.float32)

        def zloop(r, carry):
            tbuf[r, pl.ds(0, L)] = zero
            tbuf[r, pl.ds(L, L)] = zero
            return carry

        lax.fori_loop(0, ROWS_PER_TILE, zloop, 0, unroll=4)

        @pl.when(c == 1)
        def _():
            pltpu.sync_copy(tbuf, acc.at[pl.ds(row0, ROWS_PER_TILE)])

        plsc.subcore_barrier()

        base = wid * ROWS_LO + jnp.minimum(wid, N_EXTRA)
        t = ROWS_LO + jnp.where(wid < N_EXTRA, 1, 0)
        pltpu.sync_copy(e_hbm.at[0, pl.ds(base, ROWS_LO)],
                        src_v.at[pl.ds(0, ROWS_LO)])
        pltpu.sync_copy(e_hbm.at[1, pl.ds(base, ROWS_LO)],
                        dst_v.at[pl.ds(0, ROWS_LO)])

        @pl.when(wid < N_EXTRA)
        def _():
            pltpu.sync_copy(e_hbm.at[0, pl.ds(base + ROWS_LO, 1)],
                            src_v.at[pl.ds(ROWS_LO, 1)])
            pltpu.sync_copy(e_hbm.at[1, pl.ds(base + ROWS_LO, 1)],
                            dst_v.at[pl.ds(ROWS_LO, 1)])

        bufs = (rows_a, rows_b)
        sems = (sem_a, sem_b)
        pltpu.async_copy(p_sh.at[src_v.at[0]], rows_a, sem_a)

        def eloop(i, carry):
            for b in range(2):
                j = 2 * i + b
                nb = 1 - b
                pltpu.make_async_copy(p_sh.at[src_v.at[j]],
                                      bufs[b], sems[b]).wait()

                @pl.when(j + 1 < t)
                def _():
                    pltpu.async_copy(p_sh.at[src_v.at[j + 1]],
                                     bufs[nb], sems[nb])

                pltpu.sync_copy(bufs[b], acc.at[dst_v.at[j]], add=True)
            return carry

        lax.fori_loop(0, ROWS_LO // 2, eloop, 0)

        @pl.when(wid < N_EXTRA)
        def _():
            pltpu.make_async_copy(p_sh.at[src_v.at[ROWS_LO]],
                                  bufs[0], sems[0]).wait()
            pltpu.sync_copy(bufs[0], acc.at[dst_v.at[ROWS_LO]], add=True)

        plsc.subcore_barrier()

        pltpu.sync_copy(acc.at[pl.ds(row0, ROWS_PER_TILE)], tbuf)
        pltpu.sync_copy(tbuf, out_hbm.at[c, pl.ds(row0, ROWS_PER_TILE)])

    return seg_kernel(p, edges3d)


def _combine_body(h_ref, b_ref, w_ref, o_ref):
    h = jnp.maximum(h_ref[...] + b_ref[...], 0.0)
    o_ref[...] = jnp.dot(h, w_ref[...], preferred_element_type=jnp.float32)


def _combine_project(h_pre, b, W):
    return pl.pallas_call(
        _combine_body,
        grid=(N_BLKS,),
        in_specs=[pl.BlockSpec((BLK, D_HID), lambda i: (i, 0)),
                  pl.BlockSpec((1, D_HID), lambda i: (0, 0)),
                  pl.BlockSpec((D_HID, D_HID), lambda i: (0, 0))],
        out_specs=pl.BlockSpec((BLK, D_HID), lambda i: (i, 0)),
        out_shape=jax.ShapeDtypeStruct((N_ACC, D_HID), jnp.float32),
    )(h_pre, b, W)


def _pool_mlp_body(h_ref, b_ref, gid_ref,
                   wc1_ref, bc1_ref, wc2_ref, bc2_ref,
                   o_ref, sums, counts):
    i = pl.program_id(0)
    h = jnp.maximum(h_ref[...] + b_ref[...], 0.0)
    gid = gid_ref[...]
    onehot = (gid == lax.broadcasted_iota(jnp.int32, (BLK, N_GRAPHS), 1))
    onehot = onehot.astype(jnp.float32)
    dn = (((0,), (0,)), ((), ()))
    blk_sums = lax.dot_general(onehot, h, dn,
                               preferred_element_type=jnp.float32)
    blk_cnts = lax.dot_general(onehot, jnp.ones_like(h), dn,
                               preferred_element_type=jnp.float32)

    @pl.when(i == 0)
    def _():
        sums[...] = jnp.zeros_like(sums)
        counts[...] = jnp.zeros_like(counts)

    sums[...] += blk_sums
    counts[...] += blk_cnts

    @pl.when(i == N_BLKS - 1)
    def _():
        h_g = sums[...] / jnp.maximum(counts[...], 1.0)
        hid = jnp.dot(h_g, wc1_ref[...],
                      preferred_element_type=jnp.float32) + bc1_ref[...]
        hid = jnp.maximum(hid, 0.0)
        o_ref[...] = jnp.dot(hid, wc2_ref[...],
                             preferred_element_type=jnp.float32) + bc2_ref[...]


def _pool_mlp(h_pre, b, gids, Wc1, bc1, Wc2, bc2):
    return pl.pallas_call(
        _pool_mlp_body,
        grid=(N_BLKS,),
        in_specs=[pl.BlockSpec((BLK, D_HID), lambda i: (i, 0)),
                  pl.BlockSpec((1, D_HID), lambda i: (0, 0)),
                  pl.BlockSpec((BLK, 1), lambda i: (i, 0)),
                  pl.BlockSpec((D_HID, D_HID), lambda i: (0, 0)),
                  pl.BlockSpec((1, D_HID), lambda i: (0, 0)),
                  pl.BlockSpec((D_HID, N_CLS), lambda i: (0, 0)),
                  pl.BlockSpec((1, N_CLS), lambda i: (0, 0))],
        out_specs=pl.BlockSpec((N_GRAPHS, N_CLS), lambda i: (0, 0)),
        out_shape=jax.ShapeDtypeStruct((N_GRAPHS, N_CLS), jnp.float32),
        scratch_shapes=[pltpu.VMEM((N_GRAPHS, D_HID), jnp.float32),
                        pltpu.VMEM((N_GRAPHS, D_HID), jnp.float32)],
    )(h_pre, b, gids, Wc1, bc1, Wc2, bc2)


def kernel(x, edge_index, node_graph_ids, W1, b1, W2, b2, Wc1, bc1, Wc2, bc2):
    edges3d = edge_index.astype(jnp.int32).reshape(2, E_ROWS, CHUNK)
    gids = node_graph_ids.astype(jnp.int32).reshape(N_NODES, 1)

    p1 = _project(x, W1)
    parts1 = _segsum_partials(p1, edges3d)
    h1_pre = parts1[0] + parts1[1]
    p2 = _combine_project(h1_pre, b1.reshape(1, -1), W2)
    parts2 = _segsum_partials(p2, edges3d)
    h2_pre = parts2[0] + parts2[1]
    return _pool_mlp(h2_pre, b2.reshape(1, -1), gids,
                     Wc1, bc1.reshape(1, -1), Wc2, bc2.reshape(1, -1))


# speedup vs baseline: 22.6765x; 1.0936x over previous
import functools

import jax
import jax.numpy as jnp
from jax import lax
from jax.experimental import pallas as pl
from jax.experimental.pallas import tpu as pltpu
from jax.experimental.pallas import tpu_sc as plsc

N_NODES = 10000
D_HID = 32
N_GRAPHS = 64
N_CLS = 10

NC, NS, L = 2, 16, 16
NW = NC * NS
CHUNK = 128
E_ROWS = 2500
ROWS_LO = E_ROWS // NW
N_EXTRA = E_ROWS - ROWS_LO * NW
N_ACC = 10112
ROWS_PER_TILE = N_ACC // NS

BLK = 2000
N_BLKS = N_NODES // BLK


def _proj_body(x_ref, w_ref, o_ref):
    o_ref[...] = jnp.dot(x_ref[...], w_ref[...],
                         preferred_element_type=jnp.float32)


def _project(x, W):
    n, k = x.shape
    m = W.shape[1]
    return pl.pallas_call(
        _proj_body,
        grid=(n // BLK,),
        in_specs=[pl.BlockSpec((BLK, k), lambda i: (i, 0)),
                  pl.BlockSpec((k, m), lambda i: (0, 0))],
        out_specs=pl.BlockSpec((BLK, m), lambda i: (i, 0)),
        out_shape=jax.ShapeDtypeStruct((N_ACC, m), jnp.float32),
    )(x, W)


def _segsum_partials(p, edges3d):
    mesh = plsc.VectorSubcoreMesh(core_axis_name="c", subcore_axis_name="s",
                                  num_cores=NC, num_subcores=NS)

    @functools.partial(
        pl.kernel,
        out_type=jax.ShapeDtypeStruct((NC, N_ACC, D_HID), jnp.float32),
        mesh=mesh,
        scratch_types=[
            pltpu.VMEM((ROWS_LO + 1, CHUNK), jnp.int32),
            pltpu.VMEM((ROWS_LO + 1, CHUNK), jnp.int32),
            pltpu.VMEM((CHUNK, D_HID), jnp.float32),
            pltpu.VMEM((CHUNK, D_HID), jnp.float32),
            pltpu.VMEM((ROWS_PER_TILE, D_HID), jnp.float32),
            pltpu.VMEM_SHARED((N_ACC, D_HID), jnp.float32),
            pltpu.VMEM_SHARED((N_ACC, D_HID), jnp.float32),
            pltpu.SemaphoreType.DMA,
            pltpu.SemaphoreType.DMA,
        ],
        compiler_params=pltpu.CompilerParams(use_tc_tiling_on_sc=False),
    )
    def seg_kernel(p_hbm, e_hbm, out_hbm,
                   src_v, dst_v, rows_a, rows_b, tbuf, acc, p_sh,
                   sem_a, sem_b):
        c = lax.axis_index("c")
        s = lax.axis_index("s")
        wid = s * NC + c
        row0 = s * ROWS_PER_TILE

        base = wid * ROWS_LO + jnp.minimum(wid, N_EXTRA)
        t = ROWS_LO + jnp.where(wid < N_EXTRA, 1, 0)
        cp_src = pltpu.async_copy(e_hbm.at[0, pl.ds(base, ROWS_LO)],
                                  src_v.at[pl.ds(0, ROWS_LO)], sem_a)
        cp_dst = pltpu.async_copy(e_hbm.at[1, pl.ds(base, ROWS_LO)],
                                  dst_v.at[pl.ds(0, ROWS_LO)], sem_b)

        @pl.when(wid < N_EXTRA)
        def _():
            pltpu.async_copy(e_hbm.at[0, pl.ds(base + ROWS_LO, 1)],
                             src_v.at[pl.ds(ROWS_LO, 1)], sem_a)
            pltpu.async_copy(e_hbm.at[1, pl.ds(base + ROWS_LO, 1)],
                             dst_v.at[pl.ds(ROWS_LO, 1)], sem_b)

        pltpu.sync_copy(p_hbm.at[pl.ds(row0, ROWS_PER_TILE)], tbuf)
        pltpu.sync_copy(tbuf, p_sh.at[pl.ds(row0, ROWS_PER_TILE)])

        @pl.when(c == 0)
        def _():
            pltpu.sync_copy(tbuf, acc.at[pl.ds(row0, ROWS_PER_TILE)])

        zero = jnp.zeros((L,), jnp.float32)

        def zloop(r, carry):
            tbuf[r, pl.ds(0, L)] = zero
            tbuf[r, pl.ds(L, L)] = zero
            return carry

        lax.fori_loop(0, ROWS_PER_TILE, zloop, 0, unroll=4)

        @pl.when(c == 1)
        def _():
            pltpu.sync_copy(tbuf, acc.at[pl.ds(row0, ROWS_PER_TILE)])

        plsc.subcore_barrier()

        cp_src.wait()
        cp_dst.wait()

        @pl.when(wid < N_EXTRA)
        def _():
            pltpu.make_async_copy(e_hbm.at[0, pl.ds(base + ROWS_LO, 1)],
                                  src_v.at[pl.ds(ROWS_LO, 1)], sem_a).wait()
            pltpu.make_async_copy(e_hbm.at[1, pl.ds(base + ROWS_LO, 1)],
                                  dst_v.at[pl.ds(ROWS_LO, 1)], sem_b).wait()

        bufs = (rows_a, rows_b)
        sems = (sem_a, sem_b)
        pltpu.async_copy(p_sh.at[src_v.at[0]], rows_a, sem_a)

        def eloop(i, carry):
            for b in range(2):
                j = 2 * i + b
                nb = 1 - b
                pltpu.make_async_copy(p_sh.at[src_v.at[j]],
                                      bufs[b], sems[b]).wait()

                @pl.when(j + 1 < t)
                def _():
                    pltpu.async_copy(p_sh.at[src_v.at[j + 1]],
                                     bufs[nb], sems[nb])

                pltpu.sync_copy(bufs[b], acc.at[dst_v.at[j]], add=True)
            return carry

        lax.fori_loop(0, ROWS_LO // 2, eloop, 0)

        @pl.when(wid < N_EXTRA)
        def _():
            pltpu.make_async_copy(p_sh.at[src_v.at[ROWS_LO]],
                                  bufs[0], sems[0]).wait()
            pltpu.sync_copy(bufs[0], acc.at[dst_v.at[ROWS_LO]], add=True)

        plsc.subcore_barrier()

        pltpu.sync_copy(acc.at[pl.ds(row0, ROWS_PER_TILE)], tbuf)
        pltpu.sync_copy(tbuf, out_hbm.at[c, pl.ds(row0, ROWS_PER_TILE)])

    return seg_kernel(p, edges3d)


def _combine_body(parts_ref, b_ref, w_ref, o_ref):
    h = jnp.maximum(parts_ref[0] + parts_ref[1] + b_ref[...], 0.0)
    o_ref[...] = jnp.dot(h, w_ref[...], preferred_element_type=jnp.float32)


def _combine_project(parts, b, W):
    return pl.pallas_call(
        _combine_body,
        grid=(N_BLKS,),
        in_specs=[pl.BlockSpec((NC, BLK, D_HID), lambda i: (0, i, 0)),
                  pl.BlockSpec((1, D_HID), lambda i: (0, 0)),
                  pl.BlockSpec((D_HID, D_HID), lambda i: (0, 0))],
        out_specs=pl.BlockSpec((BLK, D_HID), lambda i: (i, 0)),
        out_shape=jax.ShapeDtypeStruct((N_ACC, D_HID), jnp.float32),
    )(parts, b, W)


def _pool_mlp_body(parts_ref, b_ref, gid_ref,
                   wc1_ref, bc1_ref, wc2_ref, bc2_ref,
                   o_ref, sums, counts):
    i = pl.program_id(0)
    h = jnp.maximum(parts_ref[0] + parts_ref[1] + b_ref[...], 0.0)
    gid = gid_ref[...]
    onehot = (gid == lax.broadcasted_iota(jnp.int32, (BLK, N_GRAPHS), 1))
    onehot = onehot.astype(jnp.float32)
    dn = (((0,), (0,)), ((), ()))
    blk_sums = lax.dot_general(onehot, h, dn,
                               preferred_element_type=jnp.float32)
    blk_cnts = lax.dot_general(onehot, jnp.ones_like(h), dn,
                               preferred_element_type=jnp.float32)

    @pl.when(i == 0)
    def _():
        sums[...] = jnp.zeros_like(sums)
        counts[...] = jnp.zeros_like(counts)

    sums[...] += blk_sums
    counts[...] += blk_cnts

    @pl.when(i == N_BLKS - 1)
    def _():
        h_g = sums[...] / jnp.maximum(counts[...], 1.0)
        hid = jnp.dot(h_g, wc1_ref[...],
                      preferred_element_type=jnp.float32) + bc1_ref[...]
        hid = jnp.maximum(hid, 0.0)
        o_ref[...] = jnp.dot(hid, wc2_ref[...],
                             preferred_element_type=jnp.float32) + bc2_ref[...]


def _pool_mlp(parts, b, gids, Wc1, bc1, Wc2, bc2):
    return pl.pallas_call(
        _pool_mlp_body,
        grid=(N_BLKS,),
        in_specs=[pl.BlockSpec((NC, BLK, D_HID), lambda i: (0, i, 0)),
                  pl.BlockSpec((1, D_HID), lambda i: (0, 0)),
                  pl.BlockSpec((BLK, 1), lambda i: (i, 0)),
                  pl.BlockSpec((D_HID, D_HID), lambda i: (0, 0)),
                  pl.BlockSpec((1, D_HID), lambda i: (0, 0)),
                  pl.BlockSpec((D_HID, N_CLS), lambda i: (0, 0)),
                  pl.BlockSpec((1, N_CLS), lambda i: (0, 0))],
        out_specs=pl.BlockSpec((N_GRAPHS, N_CLS), lambda i: (0, 0)),
        out_shape=jax.ShapeDtypeStruct((N_GRAPHS, N_CLS), jnp.float32),
        scratch_shapes=[pltpu.VMEM((N_GRAPHS, D_HID), jnp.float32),
                        pltpu.VMEM((N_GRAPHS, D_HID), jnp.float32)],
    )(parts, b, gids, Wc1, bc1, Wc2, bc2)


def kernel(x, edge_index, node_graph_ids, W1, b1, W2, b2, Wc1, bc1, Wc2, bc2):
    edges3d = edge_index.astype(jnp.int32).reshape(2, E_ROWS, CHUNK)
    gids = node_graph_ids.astype(jnp.int32).reshape(N_NODES, 1)

    p1 = _project(x, W1)
    parts1 = _segsum_partials(p1, edges3d)
    p2 = _combine_project(parts1, b1.reshape(1, -1), W2)
    parts2 = _segsum_partials(p2, edges3d)
    return _pool_mlp(parts2, b2.reshape(1, -1), gids,
                     Wc1, bc1.reshape(1, -1), Wc2, bc2.reshape(1, -1))


# speedup vs baseline: 24.7919x; 1.0933x over previous
import functools

import jax
import jax.numpy as jnp
from jax import lax
from jax.experimental import pallas as pl
from jax.experimental.pallas import tpu as pltpu
from jax.experimental.pallas import tpu_sc as plsc

N_NODES = 10000
D_HID = 32
N_GRAPHS = 64
N_CLS = 10

NC, NS, L = 2, 16, 16
NW = NC * NS
CHUNK = 128
E_ROWS = 2500
ROWS_LO = E_ROWS // NW
N_EXTRA = E_ROWS - ROWS_LO * NW
N_ACC = 10112
ROWS_PER_TILE = N_ACC // NS

BLK = 2000
N_BLKS = N_NODES // BLK


def _proj_body(x_ref, w_ref, o_ref):
    o_ref[...] = jnp.dot(x_ref[...], w_ref[...],
                         preferred_element_type=jnp.float32)


def _project(x, W):
    n, k = x.shape
    m = W.shape[1]
    return pl.pallas_call(
        _proj_body,
        grid=(n // BLK,),
        in_specs=[pl.BlockSpec((BLK, k), lambda i: (i, 0)),
                  pl.BlockSpec((k, m), lambda i: (0, 0))],
        out_specs=pl.BlockSpec((BLK, m), lambda i: (i, 0)),
        out_shape=jax.ShapeDtypeStruct((N_ACC, m), jnp.float32),
    )(x, W)


def _segsum_partials(p, edges3d):
    mesh = plsc.VectorSubcoreMesh(core_axis_name="c", subcore_axis_name="s",
                                  num_cores=NC, num_subcores=NS)

    @functools.partial(
        pl.kernel,
        out_type=jax.ShapeDtypeStruct((NC, N_ACC, D_HID), jnp.float32),
        mesh=mesh,
        scratch_types=[
            pltpu.VMEM((ROWS_LO + 1, CHUNK), jnp.int32),
            pltpu.VMEM((ROWS_LO + 1, CHUNK), jnp.int32),
            pltpu.VMEM((CHUNK, D_HID), jnp.float32),
            pltpu.VMEM((CHUNK, D_HID), jnp.float32),
            pltpu.VMEM((CHUNK, D_HID), jnp.float32),
            pltpu.VMEM((CHUNK, D_HID), jnp.float32),
            pltpu.VMEM((ROWS_PER_TILE, D_HID), jnp.float32),
            pltpu.VMEM_SHARED((N_ACC, D_HID), jnp.float32),
            pltpu.VMEM_SHARED((N_ACC, D_HID), jnp.float32),
            [pltpu.SemaphoreType.DMA] * 4,
            [pltpu.SemaphoreType.DMA] * 4,
            pltpu.SemaphoreType.DMA,
            pltpu.SemaphoreType.DMA,
        ],
        compiler_params=pltpu.CompilerParams(use_tc_tiling_on_sc=False),
    )
    def seg_kernel(p_hbm, e_hbm, out_hbm,
                   src_v, dst_v, rows_0, rows_1, rows_2, rows_3, tbuf,
                   acc, p_sh, gsem, ssem, sem_a, sem_b):
        c = lax.axis_index("c")
        s = lax.axis_index("s")
        wid = s * NC + c
        row0 = s * ROWS_PER_TILE

        base = wid * ROWS_LO + jnp.minimum(wid, N_EXTRA)
        t = ROWS_LO + jnp.where(wid < N_EXTRA, 1, 0)
        cp_src = pltpu.async_copy(e_hbm.at[0, pl.ds(base, ROWS_LO)],
                                  src_v.at[pl.ds(0, ROWS_LO)], sem_a)
        cp_dst = pltpu.async_copy(e_hbm.at[1, pl.ds(base, ROWS_LO)],
                                  dst_v.at[pl.ds(0, ROWS_LO)], sem_b)

        @pl.when(wid < N_EXTRA)
        def _():
            pltpu.async_copy(e_hbm.at[0, pl.ds(base + ROWS_LO, 1)],
                             src_v.at[pl.ds(ROWS_LO, 1)], sem_a)
            pltpu.async_copy(e_hbm.at[1, pl.ds(base + ROWS_LO, 1)],
                             dst_v.at[pl.ds(ROWS_LO, 1)], sem_b)

        pltpu.sync_copy(p_hbm.at[pl.ds(row0, ROWS_PER_TILE)], tbuf)
        pltpu.sync_copy(tbuf, p_sh.at[pl.ds(row0, ROWS_PER_TILE)])

        @pl.when(c == 0)
        def _():
            pltpu.sync_copy(tbuf, acc.at[pl.ds(row0, ROWS_PER_TILE)])

        zero = jnp.zeros((L,), jnp.float32)

        def zloop(r, carry):
            tbuf[r, pl.ds(0, L)] = zero
            tbuf[r, pl.ds(L, L)] = zero
            return carry

        lax.fori_loop(0, ROWS_PER_TILE, zloop, 0, unroll=4)

        @pl.when(c == 1)
        def _():
            pltpu.sync_copy(tbuf, acc.at[pl.ds(row0, ROWS_PER_TILE)])

        plsc.subcore_barrier()

        cp_src.wait()
        cp_dst.wait()

        @pl.when(wid < N_EXTRA)
        def _():
            pltpu.make_async_copy(e_hbm.at[0, pl.ds(base + ROWS_LO, 1)],
                                  src_v.at[pl.ds(ROWS_LO, 1)], sem_a).wait()
            pltpu.make_async_copy(e_hbm.at[1, pl.ds(base + ROWS_LO, 1)],
                                  dst_v.at[pl.ds(ROWS_LO, 1)], sem_b).wait()

        bufs = (rows_0, rows_1, rows_2, rows_3)

        def gwait(j, b):
            pltpu.make_async_copy(p_sh.at[src_v.at[j]],
                                  bufs[b], gsem[b]).wait()

        def swait(j, b):
            pltpu.make_async_copy(bufs[b], acc.at[dst_v.at[j]],
                                  ssem[b]).wait()

        pltpu.async_copy(p_sh.at[src_v.at[0]], bufs[0], gsem[0])
        pltpu.async_copy(p_sh.at[src_v.at[1]], bufs[1], gsem[1])

        def eloop(i, carry):
            for b in range(4):
                j = 4 * i + b
                gwait(j, b)

                @pl.when(j >= 2)
                def _():
                    swait(j - 2, (b + 2) % 4)

                pltpu.async_copy(bufs[b], acc.at[dst_v.at[j]],
                                 ssem[b], add=True)

                @pl.when(j + 2 < t)
                def _():
                    pltpu.async_copy(p_sh.at[src_v.at[j + 2]],
                                     bufs[(b + 2) % 4], gsem[(b + 2) % 4])
            return carry

        lax.fori_loop(0, ROWS_LO // 4, eloop, 0)

        gwait(76, 0)
        swait(74, 2)
        pltpu.async_copy(bufs[0], acc.at[dst_v.at[76]], ssem[0], add=True)

        @pl.when(wid < N_EXTRA)
        def _():
            pltpu.async_copy(p_sh.at[src_v.at[78]], bufs[2], gsem[2])

        gwait(77, 1)
        swait(75, 3)
        pltpu.async_copy(bufs[1], acc.at[dst_v.at[77]], ssem[1], add=True)

        @pl.when(wid < N_EXTRA)
        def _():
            gwait(78, 2)
            pltpu.async_copy(bufs[2], acc.at[dst_v.at[78]], ssem[2],
                             add=True)
            swait(78, 2)

        swait(76, 0)
        swait(77, 1)

        plsc.subcore_barrier()

        pltpu.sync_copy(acc.at[pl.ds(row0, ROWS_PER_TILE)], tbuf)
        pltpu.sync_copy(tbuf, out_hbm.at[c, pl.ds(row0, ROWS_PER_TILE)])

    return seg_kernel(p, edges3d)


def _combine_body(parts_ref, b_ref, w_ref, o_ref):
    h = jnp.maximum(parts_ref[0] + parts_ref[1] + b_ref[...], 0.0)
    o_ref[...] = jnp.dot(h, w_ref[...], preferred_element_type=jnp.float32)


def _combine_project(parts, b, W):
    return pl.pallas_call(
        _combine_body,
        grid=(N_BLKS,),
        in_specs=[pl.BlockSpec((NC, BLK, D_HID), lambda i: (0, i, 0)),
                  pl.BlockSpec((1, D_HID), lambda i: (0, 0)),
                  pl.BlockSpec((D_HID, D_HID), lambda i: (0, 0))],
        out_specs=pl.BlockSpec((BLK, D_HID), lambda i: (i, 0)),
        out_shape=jax.ShapeDtypeStruct((N_ACC, D_HID), jnp.float32),
    )(parts, b, W)


def _pool_mlp_body(parts_ref, b_ref, gid_ref,
                   wc1_ref, bc1_ref, wc2_ref, bc2_ref,
                   o_ref, sums, counts):
    i = pl.program_id(0)
    h = jnp.maximum(parts_ref[0] + parts_ref[1] + b_ref[...], 0.0)
    gid = gid_ref[...]
    onehot = (gid == lax.broadcasted_iota(jnp.int32, (BLK, N_GRAPHS), 1))
    onehot = onehot.astype(jnp.float32)
    dn = (((0,), (0,)), ((), ()))
    blk_sums = lax.dot_general(onehot, h, dn,
                               preferred_element_type=jnp.float32)
    blk_cnts = lax.dot_general(onehot, jnp.ones_like(h), dn,
                               preferred_element_type=jnp.float32)

    @pl.when(i == 0)
    def _():
        sums[...] = jnp.zeros_like(sums)
        counts[...] = jnp.zeros_like(counts)

    sums[...] += blk_sums
    counts[...] += blk_cnts

    @pl.when(i == N_BLKS - 1)
    def _():
        h_g = sums[...] / jnp.maximum(counts[...], 1.0)
        hid = jnp.dot(h_g, wc1_ref[...],
                      preferred_element_type=jnp.float32) + bc1_ref[...]
        hid = jnp.maximum(hid, 0.0)
        o_ref[...] = jnp.dot(hid, wc2_ref[...],
                             preferred_element_type=jnp.float32) + bc2_ref[...]


def _pool_mlp(parts, b, gids, Wc1, bc1, Wc2, bc2):
    return pl.pallas_call(
        _pool_mlp_body,
        grid=(N_BLKS,),
        in_specs=[pl.BlockSpec((NC, BLK, D_HID), lambda i: (0, i, 0)),
                  pl.BlockSpec((1, D_HID), lambda i: (0, 0)),
                  pl.BlockSpec((BLK, 1), lambda i: (i, 0)),
                  pl.BlockSpec((D_HID, D_HID), lambda i: (0, 0)),
                  pl.BlockSpec((1, D_HID), lambda i: (0, 0)),
                  pl.BlockSpec((D_HID, N_CLS), lambda i: (0, 0)),
                  pl.BlockSpec((1, N_CLS), lambda i: (0, 0))],
        out_specs=pl.BlockSpec((N_GRAPHS, N_CLS), lambda i: (0, 0)),
        out_shape=jax.ShapeDtypeStruct((N_GRAPHS, N_CLS), jnp.float32),
        scratch_shapes=[pltpu.VMEM((N_GRAPHS, D_HID), jnp.float32),
                        pltpu.VMEM((N_GRAPHS, D_HID), jnp.float32)],
    )(parts, b, gids, Wc1, bc1, Wc2, bc2)


def kernel(x, edge_index, node_graph_ids, W1, b1, W2, b2, Wc1, bc1, Wc2, bc2):
    edges3d = edge_index.astype(jnp.int32).reshape(2, E_ROWS, CHUNK)
    gids = node_graph_ids.astype(jnp.int32).reshape(N_NODES, 1)

    p1 = _project(x, W1)
    parts1 = _segsum_partials(p1, edges3d)
    p2 = _combine_project(parts1, b1.reshape(1, -1), W2)
    parts2 = _segsum_partials(p2, edges3d)
    return _pool_mlp(parts2, b2.reshape(1, -1), gids,
                     Wc1, bc1.reshape(1, -1), Wc2, bc2.reshape(1, -1))


# speedup vs baseline: 24.8048x; 1.0005x over previous
import functools

import jax
import jax.numpy as jnp
from jax import lax
from jax.experimental import pallas as pl
from jax.experimental.pallas import tpu as pltpu
from jax.experimental.pallas import tpu_sc as plsc

N_NODES = 10000
D_HID = 32
N_GRAPHS = 64
N_CLS = 10

NC, NS, L = 2, 16, 16
NW = NC * NS
N_EDGES = 320000
EPW = 9984
CHUNK = 128
N_EXTRA = (N_EDGES - EPW * NW) // CHUNK
EC = 312
NCH = EPW // EC
N_ACC = 10112
ROWS_PER_TILE = N_ACC // NS

BLK = 2000
N_BLKS = N_NODES // BLK


def _proj_body(x_ref, w_ref, o_ref):
    o_ref[...] = jnp.dot(x_ref[...], w_ref[...],
                         preferred_element_type=jnp.float32)


def _project(x, W):
    n, k = x.shape
    m = W.shape[1]
    return pl.pallas_call(
        _proj_body,
        grid=(n // BLK,),
        in_specs=[pl.BlockSpec((BLK, k), lambda i: (i, 0)),
                  pl.BlockSpec((k, m), lambda i: (0, 0))],
        out_specs=pl.BlockSpec((BLK, m), lambda i: (i, 0)),
        out_shape=jax.ShapeDtypeStruct((N_ACC, m), jnp.float32),
    )(x, W)


def _segsum_partials(p, edges):
    mesh = plsc.VectorSubcoreMesh(core_axis_name="c", subcore_axis_name="s",
                                  num_cores=NC, num_subcores=NS)

    @functools.partial(
        pl.kernel,
        out_type=jax.ShapeDtypeStruct((NC, N_ACC, D_HID), jnp.float32),
        mesh=mesh,
        scratch_types=[
            pltpu.VMEM((EPW + CHUNK,), jnp.int32),
            pltpu.VMEM((EPW + CHUNK,), jnp.int32),
            pltpu.VMEM((EC, D_HID), jnp.float32),
            pltpu.VMEM((EC, D_HID), jnp.float32),
            pltpu.VMEM((EC, D_HID), jnp.float32),
            pltpu.VMEM((EC, D_HID), jnp.float32),
            pltpu.VMEM((ROWS_PER_TILE, D_HID), jnp.float32),
            pltpu.VMEM_SHARED((N_ACC, D_HID), jnp.float32),
            pltpu.VMEM_SHARED((N_ACC, D_HID), jnp.float32),
            [pltpu.SemaphoreType.DMA] * 4,
            [pltpu.SemaphoreType.DMA] * 4,
            pltpu.SemaphoreType.DMA,
            pltpu.SemaphoreType.DMA,
        ],
        compiler_params=pltpu.CompilerParams(use_tc_tiling_on_sc=False),
    )
    def seg_kernel(p_hbm, e_hbm, out_hbm,
                   src_v, dst_v, rows_0, rows_1, rows_2, rows_3, tbuf,
                   acc, p_sh, gsem, ssem, sem_a, sem_b):
        c = lax.axis_index("c")
        s = lax.axis_index("s")
        wid = s * NC + c
        row0 = s * ROWS_PER_TILE

        base = wid * EPW + jnp.minimum(wid, N_EXTRA) * CHUNK
        cp_src = pltpu.async_copy(e_hbm.at[0, pl.ds(base, EPW)],
                                  src_v.at[pl.ds(0, EPW)], sem_a)
        cp_dst = pltpu.async_copy(e_hbm.at[1, pl.ds(base, EPW)],
                                  dst_v.at[pl.ds(0, EPW)], sem_b)

        @pl.when(wid < N_EXTRA)
        def _():
            pltpu.async_copy(e_hbm.at[0, pl.ds(base + EPW, CHUNK)],
                             src_v.at[pl.ds(EPW, CHUNK)], sem_a)
            pltpu.async_copy(e_hbm.at[1, pl.ds(base + EPW, CHUNK)],
                             dst_v.at[pl.ds(EPW, CHUNK)], sem_b)

        pltpu.sync_copy(p_hbm.at[pl.ds(row0, ROWS_PER_TILE)], tbuf)
        pltpu.sync_copy(tbuf, p_sh.at[pl.ds(row0, ROWS_PER_TILE)])

        @pl.when(c == 0)
        def _():
            pltpu.sync_copy(tbuf, acc.at[pl.ds(row0, ROWS_PER_TILE)])

        zero = jnp.zeros((L,), jnp.float32)

        def zloop(r, carry):
            tbuf[r, pl.ds(0, L)] = zero
            tbuf[r, pl.ds(L, L)] = zero
            return carry

        lax.fori_loop(0, ROWS_PER_TILE, zloop, 0, unroll=4)

        @pl.when(c == 1)
        def _():
            pltpu.sync_copy(tbuf, acc.at[pl.ds(row0, ROWS_PER_TILE)])

        plsc.subcore_barrier()

        cp_src.wait()
        cp_dst.wait()

        @pl.when(wid < N_EXTRA)
        def _():
            pltpu.make_async_copy(e_hbm.at[0, pl.ds(base + EPW, CHUNK)],
                                  src_v.at[pl.ds(EPW, CHUNK)], sem_a).wait()
            pltpu.make_async_copy(e_hbm.at[1, pl.ds(base + EPW, CHUNK)],
                                  dst_v.at[pl.ds(EPW, CHUNK)], sem_b).wait()

        bufs = (rows_0, rows_1, rows_2, rows_3)

        def sidx(j):
            return src_v.at[pl.ds(j * EC, EC)]

        def didx(j):
            return dst_v.at[pl.ds(j * EC, EC)]

        def gwait(j, b):
            pltpu.make_async_copy(p_sh.at[sidx(j)], bufs[b], gsem[b]).wait()

        def swait(j, b):
            pltpu.make_async_copy(bufs[b], acc.at[didx(j)], ssem[b]).wait()

        pltpu.async_copy(p_sh.at[sidx(0)], bufs[0], gsem[0])
        pltpu.async_copy(p_sh.at[sidx(1)], bufs[1], gsem[1])

        for j in range(NCH):
            b = j % 4
            gwait(j, b)
            if j >= 2:
                swait(j - 2, (j - 2) % 4)
            pltpu.async_copy(bufs[b], acc.at[didx(j)], ssem[b], add=True)
            if j + 2 < NCH:
                pltpu.async_copy(p_sh.at[sidx(j + 2)],
                                 bufs[(j + 2) % 4], gsem[(j + 2) % 4])

        @pl.when(wid < N_EXTRA)
        def _():
            xs = src_v.at[pl.ds(EPW, CHUNK)]
            xd = dst_v.at[pl.ds(EPW, CHUNK)]
            xbuf = rows_0.at[pl.ds(0, CHUNK)]
            pltpu.async_copy(p_sh.at[xs], xbuf, gsem[0])
            pltpu.make_async_copy(p_sh.at[xs], xbuf, gsem[0]).wait()
            pltpu.async_copy(xbuf, acc.at[xd], ssem[0], add=True)
            pltpu.make_async_copy(xbuf, acc.at[xd], ssem[0]).wait()

        swait(NCH - 2, (NCH - 2) % 4)
        swait(NCH - 1, (NCH - 1) % 4)

        plsc.subcore_barrier()

        pltpu.sync_copy(acc.at[pl.ds(row0, ROWS_PER_TILE)], tbuf)
        pltpu.sync_copy(tbuf, out_hbm.at[c, pl.ds(row0, ROWS_PER_TILE)])

    return seg_kernel(p, edges)


def _combine_body(parts_ref, b_ref, w_ref, o_ref):
    h = jnp.maximum(parts_ref[0] + parts_ref[1] + b_ref[...], 0.0)
    o_ref[...] = jnp.dot(h, w_ref[...], preferred_element_type=jnp.float32)


def _combine_project(parts, b, W):
    return pl.pallas_call(
        _combine_body,
        grid=(N_BLKS,),
        in_specs=[pl.BlockSpec((NC, BLK, D_HID), lambda i: (0, i, 0)),
                  pl.BlockSpec((1, D_HID), lambda i: (0, 0)),
                  pl.BlockSpec((D_HID, D_HID), lambda i: (0, 0))],
        out_specs=pl.BlockSpec((BLK, D_HID), lambda i: (i, 0)),
        out_shape=jax.ShapeDtypeStruct((N_ACC, D_HID), jnp.float32),
    )(parts, b, W)


def _pool_mlp_body(parts_ref, b_ref, gid_ref,
                   wc1_ref, bc1_ref, wc2_ref, bc2_ref,
                   o_ref, sums, counts):
    i = pl.program_id(0)
    h = jnp.maximum(parts_ref[0] + parts_ref[1] + b_ref[...], 0.0)
    gid = gid_ref[...]
    onehot = (gid == lax.broadcasted_iota(jnp.int32, (BLK, N_GRAPHS), 1))
    onehot = onehot.astype(jnp.float32)
    dn = (((0,), (0,)), ((), ()))
    blk_sums = lax.dot_general(onehot, h, dn,
                               preferred_element_type=jnp.float32)
    blk_cnts = lax.dot_general(onehot, jnp.ones_like(h), dn,
                               preferred_element_type=jnp.float32)

    @pl.when(i == 0)
    def _():
        sums[...] = jnp.zeros_like(sums)
        counts[...] = jnp.zeros_like(counts)

    sums[...] += blk_sums
    counts[...] += blk_cnts

    @pl.when(i == N_BLKS - 1)
    def _():
        h_g = sums[...] / jnp.maximum(counts[...], 1.0)
        hid = jnp.dot(h_g, wc1_ref[...],
                      preferred_element_type=jnp.float32) + bc1_ref[...]
        hid = jnp.maximum(hid, 0.0)
        o_ref[...] = jnp.dot(hid, wc2_ref[...],
                             preferred_element_type=jnp.float32) + bc2_ref[...]


def _pool_mlp(parts, b, gids, Wc1, bc1, Wc2, bc2):
    return pl.pallas_call(
        _pool_mlp_body,
        grid=(N_BLKS,),
        in_specs=[pl.BlockSpec((NC, BLK, D_HID), lambda i: (0, i, 0)),
                  pl.BlockSpec((1, D_HID), lambda i: (0, 0)),
                  pl.BlockSpec((BLK, 1), lambda i: (i, 0)),
                  pl.BlockSpec((D_HID, D_HID), lambda i: (0, 0)),
                  pl.BlockSpec((1, D_HID), lambda i: (0, 0)),
                  pl.BlockSpec((D_HID, N_CLS), lambda i: (0, 0)),
                  pl.BlockSpec((1, N_CLS), lambda i: (0, 0))],
        out_specs=pl.BlockSpec((N_GRAPHS, N_CLS), lambda i: (0, 0)),
        out_shape=jax.ShapeDtypeStruct((N_GRAPHS, N_CLS), jnp.float32),
        scratch_shapes=[pltpu.VMEM((N_GRAPHS, D_HID), jnp.float32),
                        pltpu.VMEM((N_GRAPHS, D_HID), jnp.float32)],
    )(parts, b, gids, Wc1, bc1, Wc2, bc2)


def kernel(x, edge_index, node_graph_ids, W1, b1, W2, b2, Wc1, bc1, Wc2, bc2):
    edges = edge_index.astype(jnp.int32)
    gids = node_graph_ids.astype(jnp.int32).reshape(N_NODES, 1)

    p1 = _project(x, W1)
    parts1 = _segsum_partials(p1, edges)
    p2 = _combine_project(parts1, b1.reshape(1, -1), W2)
    parts2 = _segsum_partials(p2, edges)
    return _pool_mlp(parts2, b2.reshape(1, -1), gids,
                     Wc1, bc1.reshape(1, -1), Wc2, bc2.reshape(1, -1))
